# Initial kernel scaffold; baseline (speedup 1.0000x reference)
import jax
import jax.numpy as jnp
from jax.experimental import pallas as pl


def kernel(inputs, bxyz_indx, W, b):
    raise NotImplementedError("write your pallas kernel here")


# speedup vs baseline: 1.6087x; 1.6087x over previous
import functools
import jax
import jax.numpy as jnp
from jax.experimental import pallas as pl
from jax.experimental.pallas import tpu as pltpu

N = 320000
C_IN = 128
UNITS = 64
NKEYS = 16 * 16 * 16 * 16

BM = 1280


def _matmul_body(x_ref, w_ref, b_ref, o_ref):
    o_ref[...] = (
        jnp.dot(x_ref[...], w_ref[...], preferred_element_type=jnp.float32)
        + b_ref[...]
    )


def _matmul(inputs, W, b2d):
    return pl.pallas_call(
        _matmul_body,
        grid=(N // BM,),
        in_specs=[
            pl.BlockSpec((BM, C_IN), lambda i: (i, 0)),
            pl.BlockSpec((C_IN, UNITS), lambda i: (0, 0)),
            pl.BlockSpec((1, UNITS), lambda i: (0, 0)),
        ],
        out_specs=pl.BlockSpec((BM, UNITS), lambda i: (i, 0)),
        out_shape=jax.ShapeDtypeStruct((N, UNITS), jnp.float32),
    )(inputs, W, b2d)


def kernel(inputs, bxyz_indx, W, b):
    x = _matmul(inputs, W, b.reshape(1, UNITS))
    key = (
        ((bxyz_indx[:, 0] * 16 + bxyz_indx[:, 1]) * 16 + bxyz_indx[:, 2]) * 16
        + bxyz_indx[:, 3]
    )
    table = jax.ops.segment_max(x, key, num_segments=NKEYS)
    gathered = jnp.take(table, key, axis=0)
    return jnp.concatenate([x, gathered], axis=1)
